# Initial kernel scaffold; baseline (speedup 1.0000x reference)
import functools
import math

import jax
import jax.numpy as jnp
from jax import lax
from jax.experimental import pallas as pl
from jax.experimental.pallas import tpu as pltpu
from jax.experimental.pallas import tpu_sc as plsc

_B, _N, _D, _C, _H, _K = 2, 2048, 128, 3, 256, 16
_TPAD = 144
_RA = 256
_RC = 256

_NC, _NS = 2, 16
_NW = _NC * _NS
_E = _B * _N * _K
_EPW = _E // _NW
_CHUNK = 128


def _knn_body(coords_ref, coordst_ref, idx_ref):
    b = pl.program_id(0)
    xi = coords_ref[0]
    xt = coordst_ref[0]
    d0 = xi[:, 0:1] - xt[0:1, :]
    acc = d0 * d0
    d1 = xi[:, 1:2] - xt[1:2, :]
    acc = acc + d1 * d1
    d2 = xi[:, 2:3] - xt[2:3, :]
    acc = acc + d2 * d2

    iota = lax.broadcasted_iota(jnp.int32, (_RA, _N), 1)
    cols = []
    dist = acc
    for _ in range(_K):
        m = jnp.min(dist, axis=1, keepdims=True)
        cand = jnp.where(dist == m, iota, _N)
        j = jnp.min(cand, axis=1, keepdims=True)
        cols.append(j)
        dist = jnp.where(iota == j, jnp.float32(jnp.inf), dist)
    idx = jnp.concatenate(cols, axis=1)
    idx_ref[0] = idx + b * _N


def _knn_call(coords, coordst):
    return pl.pallas_call(
        _knn_body,
        grid=(_B, _N // _RA),
        in_specs=[
            pl.BlockSpec((1, _RA, _C), lambda b, i: (b, i, 0)),
            pl.BlockSpec((1, _C, _N), lambda b, i: (b, 0, 0)),
        ],
        out_specs=pl.BlockSpec((1, _RA, _K), lambda b, i: (b, i, 0)),
        out_shape=jax.ShapeDtypeStruct((_B, _N, _K), jnp.int32),
    )(coords, coordst)


def _gather_body(t_hbm, idx_hbm, out_hbm, idx_v, rows_v, sem):
    wid = lax.axis_index("s") * _NC + lax.axis_index("c")
    row0 = wid * (_EPW // _CHUNK)
    pltpu.sync_copy(idx_hbm.at[pl.ds(row0, _EPW // _CHUNK)], idx_v)

    def body(ci, carry):
        pltpu.async_copy(t_hbm.at[idx_v.at[ci]], rows_v, sem).wait()
        pltpu.sync_copy(
            rows_v, out_hbm.at[pl.ds((row0 + ci) * _CHUNK, _CHUNK)])
        return carry

    lax.fori_loop(0, _EPW // _CHUNK, body, 0)


def _gather_call(t_flat, idx2d):
    mesh = plsc.VectorSubcoreMesh(
        core_axis_name="c", subcore_axis_name="s",
        num_cores=_NC, num_subcores=_NS)
    f = pl.kernel(
        _gather_body,
        out_type=jax.ShapeDtypeStruct((_E, _TPAD), jnp.float32),
        mesh=mesh,
        scratch_types=[
            pltpu.VMEM((_EPW // _CHUNK, _CHUNK), jnp.int32),
            pltpu.VMEM((_CHUNK, _TPAD), jnp.float32),
            pltpu.SemaphoreType.DMA,
        ],
    )
    return f(t_flat, idx2d)


def _mlp_body(t_ref, g_ref, wi_ref, wg_ref, b1_ref, w2_ref, b2_ref,
              w3_ref, b3_ref, wl_ref, bl_ref, out_ref):
    t = t_ref[0]
    g = g_ref[0]
    a_i = jnp.dot(t, wi_ref[...], preferred_element_type=jnp.float32)
    a_i = a_i + b1_ref[...]
    a_j = jnp.dot(g, wg_ref[...], preferred_element_type=jnp.float32)
    a_i_rep = jnp.broadcast_to(
        a_i[:, None, :], (_RC, _K, _H)).reshape(_RC * _K, _H)
    h1 = jnp.maximum(a_j + a_i_rep, 0.0)
    h2 = jnp.dot(h1, w2_ref[...], preferred_element_type=jnp.float32)
    h2 = jnp.maximum(h2 + b2_ref[...], 0.0)
    kraw = jnp.sum(h2 * w3_ref[...], axis=1, keepdims=True) + b3_ref[...]
    kw = kraw * (1.0 / math.sqrt(_K))
    msg = kw * g[:, :_D]
    agg = jnp.sum(msg.reshape(_RC, _K, _D), axis=1)
    u_blk = t[:, :_D]
    out = jnp.dot(u_blk, wl_ref[...], preferred_element_type=jnp.float32)
    out = out + bl_ref[...] + agg + u_blk
    out_ref[0] = jnp.maximum(out, 0.0)


def _mlp_call(t3, g3, wi, wg, b1, w2, b2, w3r, b3, wl, bl):
    full = lambda s: pl.BlockSpec(s, lambda b, i: tuple(0 for _ in s))
    return pl.pallas_call(
        _mlp_body,
        grid=(_B, _N // _RC),
        in_specs=[
            pl.BlockSpec((1, _RC, _TPAD), lambda b, i: (b, i, 0)),
            pl.BlockSpec((1, _RC * _K, _TPAD), lambda b, i: (b, i, 0)),
            full((_TPAD, _H)),
            full((_TPAD, _H)),
            full((1, _H)),
            full((_H, _H)),
            full((1, _H)),
            full((1, _H)),
            full((1, 1)),
            full((_D, _D)),
            full((1, _D)),
        ],
        out_specs=pl.BlockSpec((1, _RC, _D), lambda b, i: (b, i, 0)),
        out_shape=jax.ShapeDtypeStruct((_B, _N, _D), jnp.float32),
    )(t3, g3, wi, wg, b1, w2, b2, w3r, b3, wl, bl)


def kernel(u, coords, W_local, b_local, W1, b1, W2, b2, W3, b3):
    b, n, d = u.shape
    c = coords.shape[-1]
    pad = jnp.zeros((b, n, _TPAD - d - c), jnp.float32)
    t3 = jnp.concatenate([u, coords, pad], axis=-1)
    coordst = jnp.swapaxes(coords, 1, 2)

    idx = _knn_call(coords, coordst)
    idx2d = idx.reshape(_E // _CHUNK, _CHUNK)

    t_flat = t3.reshape(b * n, _TPAD)
    g = _gather_call(t_flat, idx2d)
    g3 = g.reshape(b, n * _K, _TPAD)

    zpad = jnp.zeros((_TPAD - d - c, _H), jnp.float32)
    wi = jnp.concatenate([W1[2 * c:2 * c + d], W1[0:c], zpad], axis=0)
    wg = jnp.concatenate([W1[2 * c + d:], W1[c:2 * c], zpad], axis=0)

    out = _mlp_call(
        t3, g3, wi, wg,
        b1.reshape(1, _H), W2, b2.reshape(1, _H),
        W3.reshape(1, _H), b3.reshape(1, 1),
        W_local, b_local.reshape(1, _D))
    return out


# speedup vs baseline: 8.1161x; 8.1161x over previous
import functools
import math

import jax
import jax.numpy as jnp
from jax import lax
from jax.experimental import pallas as pl
from jax.experimental.pallas import tpu as pltpu
from jax.experimental.pallas import tpu_sc as plsc

_B, _N, _D, _C, _H, _K = 2, 2048, 128, 3, 256, 16
_TPAD = 144
_RA = 256
_RC = 256

_NC, _NS = 2, 16
_NW = _NC * _NS
_E = _B * _N * _K
_EPW = _E // _NW
_CHUNK = 128


def _knn_body(coords_ref, coordst_ref, idx_ref):
    b = pl.program_id(0)
    xi = coords_ref[0]
    xt = coordst_ref[0]
    d0 = xi[:, 0:1] - xt[0:1, :]
    acc = d0 * d0
    d1 = xi[:, 1:2] - xt[1:2, :]
    acc = acc + d1 * d1
    d2 = xi[:, 2:3] - xt[2:3, :]
    acc = acc + d2 * d2

    iota = lax.broadcasted_iota(jnp.int32, (_RA, _N), 1)
    cols = []
    dist = acc
    for _ in range(_K):
        m = jnp.min(dist, axis=1, keepdims=True)
        cand = jnp.where(dist == m, iota, _N)
        j = jnp.min(cand, axis=1, keepdims=True)
        cols.append(j)
        dist = jnp.where(iota == j, jnp.float32(jnp.inf), dist)
    idx = jnp.concatenate(cols, axis=1)
    idx_ref[0] = idx + b * _N


def _knn_call(coords, coordst):
    return pl.pallas_call(
        _knn_body,
        grid=(_B, _N // _RA),
        in_specs=[
            pl.BlockSpec((1, _RA, _C), lambda b, i: (b, i, 0)),
            pl.BlockSpec((1, _C, _N), lambda b, i: (b, 0, 0)),
        ],
        out_specs=pl.BlockSpec((1, _RA, _K), lambda b, i: (b, i, 0)),
        out_shape=jax.ShapeDtypeStruct((_B, _N, _K), jnp.int32),
    )(coords, coordst)


def _gather_body(t_hbm, idx_hbm, out_hbm, idx_v, rows_v, sem):
    wid = lax.axis_index("s") * _NC + lax.axis_index("c")
    row0 = wid * (_EPW // _CHUNK)
    pltpu.sync_copy(idx_hbm.at[pl.ds(row0, _EPW // _CHUNK)], idx_v)

    def body(ci, carry):
        pltpu.async_copy(t_hbm.at[idx_v.at[ci]], rows_v, sem).wait()
        pltpu.sync_copy(
            rows_v, out_hbm.at[pl.ds((row0 + ci) * _CHUNK, _CHUNK)])
        return carry

    lax.fori_loop(0, _EPW // _CHUNK, body, 0)


def _gather_call(t_flat, idx2d):
    mesh = plsc.VectorSubcoreMesh(
        core_axis_name="c", subcore_axis_name="s",
        num_cores=_NC, num_subcores=_NS)
    f = pl.kernel(
        _gather_body,
        out_type=jax.ShapeDtypeStruct((_E, _TPAD), jnp.float32),
        mesh=mesh,
        compiler_params=pltpu.CompilerParams(use_tc_tiling_on_sc=False),
        scratch_types=[
            pltpu.VMEM((_EPW // _CHUNK, _CHUNK), jnp.int32),
            pltpu.VMEM((_CHUNK, _TPAD), jnp.float32),
            pltpu.SemaphoreType.DMA,
        ],
    )
    return f(t_flat, idx2d)


def _mlp_body(t_ref, g_ref, wi_ref, wg_ref, b1_ref, w2_ref, b2_ref,
              w3_ref, b3_ref, wl_ref, bl_ref, out_ref):
    t = t_ref[0]
    g = g_ref[0]
    a_i = jnp.dot(t, wi_ref[...], preferred_element_type=jnp.float32)
    a_i = a_i + b1_ref[...]
    a_j = jnp.dot(g, wg_ref[...], preferred_element_type=jnp.float32)
    a_i_rep = jnp.broadcast_to(
        a_i[:, None, :], (_RC, _K, _H)).reshape(_RC * _K, _H)
    h1 = jnp.maximum(a_j + a_i_rep, 0.0)
    h2 = jnp.dot(h1, w2_ref[...], preferred_element_type=jnp.float32)
    h2 = jnp.maximum(h2 + b2_ref[...], 0.0)
    kraw = jnp.sum(h2 * w3_ref[...], axis=1, keepdims=True) + b3_ref[...]
    kw = kraw * (1.0 / math.sqrt(_K))
    msg = kw * g[:, :_D]
    agg = jnp.sum(msg.reshape(_RC, _K, _D), axis=1)
    u_blk = t[:, :_D]
    out = jnp.dot(u_blk, wl_ref[...], preferred_element_type=jnp.float32)
    out = out + bl_ref[...] + agg + u_blk
    out_ref[0] = jnp.maximum(out, 0.0)


def _mlp_call(t3, g3, wi, wg, b1, w2, b2, w3r, b3, wl, bl):
    full = lambda s: pl.BlockSpec(s, lambda b, i: tuple(0 for _ in s))
    return pl.pallas_call(
        _mlp_body,
        grid=(_B, _N // _RC),
        in_specs=[
            pl.BlockSpec((1, _RC, _TPAD), lambda b, i: (b, i, 0)),
            pl.BlockSpec((1, _RC * _K, _TPAD), lambda b, i: (b, i, 0)),
            full((_TPAD, _H)),
            full((_TPAD, _H)),
            full((1, _H)),
            full((_H, _H)),
            full((1, _H)),
            full((1, _H)),
            full((1, 1)),
            full((_D, _D)),
            full((1, _D)),
        ],
        out_specs=pl.BlockSpec((1, _RC, _D), lambda b, i: (b, i, 0)),
        out_shape=jax.ShapeDtypeStruct((_B, _N, _D), jnp.float32),
    )(t3, g3, wi, wg, b1, w2, b2, w3r, b3, wl, bl)


def kernel(u, coords, W_local, b_local, W1, b1, W2, b2, W3, b3):
    b, n, d = u.shape
    c = coords.shape[-1]
    pad = jnp.zeros((b, n, _TPAD - d - c), jnp.float32)
    t3 = jnp.concatenate([u, coords, pad], axis=-1)
    coordst = jnp.swapaxes(coords, 1, 2)

    idx = _knn_call(coords, coordst)
    idx2d = idx.reshape(_E // _CHUNK, _CHUNK)

    t_flat = t3.reshape(b * n, _TPAD)
    g = _gather_call(t_flat, idx2d)
    g3 = g.reshape(b, n * _K, _TPAD)

    zpad = jnp.zeros((_TPAD - d - c, _H), jnp.float32)
    wi = jnp.concatenate([W1[2 * c:2 * c + d], W1[0:c], zpad], axis=0)
    wg = jnp.concatenate([W1[2 * c + d:], W1[c:2 * c], zpad], axis=0)

    out = _mlp_call(
        t3, g3, wi, wg,
        b1.reshape(1, _H), W2, b2.reshape(1, _H),
        W3.reshape(1, _H), b3.reshape(1, 1),
        W_local, b_local.reshape(1, _D))
    return out


# speedup vs baseline: 8.8591x; 1.0915x over previous
import functools
import math

import jax
import jax.numpy as jnp
from jax import lax
from jax.experimental import pallas as pl
from jax.experimental.pallas import tpu as pltpu
from jax.experimental.pallas import tpu_sc as plsc

_B, _N, _D, _C, _H, _K = 2, 2048, 128, 3, 256, 16
_TPAD = 144
_RA = 256
_RC = 256

_NC, _NS = 2, 16
_NW = _NC * _NS
_E = _B * _N * _K
_EPW = _E // _NW
_CHUNK = 128


def _knn_body(coords_ref, coordst_ref, idx_ref):
    b = pl.program_id(0)
    xi = coords_ref[0]
    xt = coordst_ref[0]
    d0 = xi[:, 0:1] - xt[0:1, :]
    acc = d0 * d0
    d1 = xi[:, 1:2] - xt[1:2, :]
    acc = acc + d1 * d1
    d2 = xi[:, 2:3] - xt[2:3, :]
    acc = acc + d2 * d2

    iota = lax.broadcasted_iota(jnp.int32, (_RA, _N), 1).astype(jnp.float32)
    big = jnp.float32(_N)
    cols = []
    dist = acc
    for _ in range(_K):
        m = jnp.min(dist, axis=1, keepdims=True)
        cand = jnp.where(dist == m, iota, big)
        j = jnp.min(cand, axis=1, keepdims=True)
        cols.append(j)
        dist = jnp.where(cand == j, jnp.float32(jnp.inf), dist)
    idx = jnp.concatenate(cols, axis=1).astype(jnp.int32)
    idx_ref[0] = idx + b * _N


def _knn_call(coords, coordst):
    return pl.pallas_call(
        _knn_body,
        grid=(_B, _N // _RA),
        in_specs=[
            pl.BlockSpec((1, _RA, _C), lambda b, i: (b, i, 0)),
            pl.BlockSpec((1, _C, _N), lambda b, i: (b, 0, 0)),
        ],
        out_specs=pl.BlockSpec((1, _RA, _K), lambda b, i: (b, i, 0)),
        out_shape=jax.ShapeDtypeStruct((_B, _N, _K), jnp.int32),
    )(coords, coordst)


def _gather_body(t_hbm, idx_hbm, out_hbm, idx_v, rows_v, sem):
    wid = lax.axis_index("s") * _NC + lax.axis_index("c")
    row0 = wid * (_EPW // _CHUNK)
    pltpu.sync_copy(idx_hbm.at[pl.ds(row0, _EPW // _CHUNK)], idx_v)

    def body(ci, carry):
        pltpu.async_copy(t_hbm.at[idx_v.at[ci]], rows_v, sem).wait()
        pltpu.sync_copy(
            rows_v, out_hbm.at[pl.ds((row0 + ci) * _CHUNK, _CHUNK)])
        return carry

    lax.fori_loop(0, _EPW // _CHUNK, body, 0)


def _gather_call(t_flat, idx2d):
    mesh = plsc.VectorSubcoreMesh(
        core_axis_name="c", subcore_axis_name="s",
        num_cores=_NC, num_subcores=_NS)
    f = pl.kernel(
        _gather_body,
        out_type=jax.ShapeDtypeStruct((_E, _TPAD), jnp.float32),
        mesh=mesh,
        compiler_params=pltpu.CompilerParams(use_tc_tiling_on_sc=False),
        scratch_types=[
            pltpu.VMEM((_EPW // _CHUNK, _CHUNK), jnp.int32),
            pltpu.VMEM((_CHUNK, _TPAD), jnp.float32),
            pltpu.SemaphoreType.DMA,
        ],
    )
    return f(t_flat, idx2d)


def _mlp_body(t_ref, g_ref, wi_ref, wg_ref, b1_ref, w2_ref, b2_ref,
              w3_ref, b3_ref, wl_ref, bl_ref, out_ref):
    t = t_ref[0]
    bf = jnp.bfloat16
    wg = wg_ref[...].astype(bf)
    w2 = w2_ref[...].astype(bf)
    a_i = jnp.dot(t.astype(bf), wi_ref[...].astype(bf),
                  preferred_element_type=jnp.float32)
    a_i = a_i + b1_ref[...]
    sub = 4
    rows = _RC * _K // sub
    nod = _RC // sub
    agg_parts = []
    for s in range(sub):
        gs = g_ref[0, s * rows:(s + 1) * rows, :]
        a_j = jnp.dot(gs.astype(bf), wg, preferred_element_type=jnp.float32)
        ai_s = a_i[s * nod:(s + 1) * nod]
        a_i_rep = jnp.broadcast_to(
            ai_s[:, None, :], (nod, _K, _H)).reshape(rows, _H)
        h1 = jnp.maximum(a_j + a_i_rep, 0.0)
        h2 = jnp.dot(h1.astype(bf), w2, preferred_element_type=jnp.float32)
        h2 = jnp.maximum(h2 + b2_ref[...], 0.0)
        kraw = jnp.sum(h2 * w3_ref[...], axis=1, keepdims=True) + b3_ref[...]
        kw = kraw * (1.0 / math.sqrt(_K))
        msg = kw * gs[:, :_D]
        agg_parts.append(jnp.sum(msg.reshape(nod, _K, _D), axis=1))
    agg = jnp.concatenate(agg_parts, axis=0)
    u_blk = t[:, :_D]
    out = jnp.dot(u_blk, wl_ref[...], preferred_element_type=jnp.float32)
    out = out + bl_ref[...] + agg + u_blk
    out_ref[0] = jnp.maximum(out, 0.0)


def _mlp_call(t3, g3, wi, wg, b1, w2, b2, w3r, b3, wl, bl):
    full = lambda s: pl.BlockSpec(s, lambda b, i: tuple(0 for _ in s))
    return pl.pallas_call(
        _mlp_body,
        grid=(_B, _N // _RC),
        in_specs=[
            pl.BlockSpec((1, _RC, _TPAD), lambda b, i: (b, i, 0)),
            pl.BlockSpec((1, _RC * _K, _TPAD), lambda b, i: (b, i, 0)),
            full((_TPAD, _H)),
            full((_TPAD, _H)),
            full((1, _H)),
            full((_H, _H)),
            full((1, _H)),
            full((1, _H)),
            full((1, 1)),
            full((_D, _D)),
            full((1, _D)),
        ],
        out_specs=pl.BlockSpec((1, _RC, _D), lambda b, i: (b, i, 0)),
        out_shape=jax.ShapeDtypeStruct((_B, _N, _D), jnp.float32),
    )(t3, g3, wi, wg, b1, w2, b2, w3r, b3, wl, bl)


def kernel(u, coords, W_local, b_local, W1, b1, W2, b2, W3, b3):
    b, n, d = u.shape
    c = coords.shape[-1]
    pad = jnp.zeros((b, n, _TPAD - d - c), jnp.float32)
    t3 = jnp.concatenate([u, coords, pad], axis=-1)
    coordst = jnp.swapaxes(coords, 1, 2)

    idx = _knn_call(coords, coordst)
    idx2d = idx.reshape(_E // _CHUNK, _CHUNK)

    t_flat = t3.reshape(b * n, _TPAD)
    g = _gather_call(t_flat, idx2d)
    g3 = g.reshape(b, n * _K, _TPAD)

    zpad = jnp.zeros((_TPAD - d - c, _H), jnp.float32)
    wi = jnp.concatenate([W1[2 * c:2 * c + d], W1[0:c], zpad], axis=0)
    wg = jnp.concatenate([W1[2 * c + d:], W1[c:2 * c], zpad], axis=0)

    out = _mlp_call(
        t3, g3, wi, wg,
        b1.reshape(1, _H), W2, b2.reshape(1, _H),
        W3.reshape(1, _H), b3.reshape(1, 1),
        W_local, b_local.reshape(1, _D))
    return out


# speedup vs baseline: 11.2529x; 1.2702x over previous
import functools
import math

import jax
import jax.numpy as jnp
from jax import lax
from jax.experimental import pallas as pl
from jax.experimental.pallas import tpu as pltpu
from jax.experimental.pallas import tpu_sc as plsc

_B, _N, _D, _C, _H, _K = 2, 2048, 128, 3, 256, 16
_TPAD = 144
_RA = 256
_RC = 256

_NC, _NS = 2, 16
_NW = _NC * _NS
_E = _B * _N * _K
_EPW = _E // _NW
_CHUNK = 128


def _knn_body(coords_ref, coordst_ref, idx_ref):
    b = pl.program_id(0)
    xi = coords_ref[0]
    xt = coordst_ref[0]
    d0 = xi[:, 0:1] - xt[0:1, :]
    acc = d0 * d0
    d1 = xi[:, 1:2] - xt[1:2, :]
    acc = acc + d1 * d1
    d2 = xi[:, 2:3] - xt[2:3, :]
    acc = acc + d2 * d2

    iota = lax.broadcasted_iota(jnp.int32, (_RA, _N), 1).astype(jnp.float32)
    big = jnp.float32(_N)
    cols = []
    dist = acc
    for _ in range(_K):
        m = jnp.min(dist, axis=1, keepdims=True)
        cand = jnp.where(dist == m, iota, big)
        j = jnp.min(cand, axis=1, keepdims=True)
        cols.append(j)
        dist = jnp.where(cand == j, jnp.float32(jnp.inf), dist)
    idx = jnp.concatenate(cols, axis=1).astype(jnp.int32)
    idx_ref[0] = idx.T + b * _N


def _knn_call(coords, coordst):
    return pl.pallas_call(
        _knn_body,
        grid=(_B, _N // _RA),
        in_specs=[
            pl.BlockSpec((1, _RA, _C), lambda b, i: (b, i, 0)),
            pl.BlockSpec((1, _C, _N), lambda b, i: (b, 0, 0)),
        ],
        out_specs=pl.BlockSpec((1, _K, _RA), lambda b, i: (b, 0, i)),
        out_shape=jax.ShapeDtypeStruct((_B, _K, _N), jnp.int32),
    )(coords, coordst)


def _gather_body(t_hbm, idx_hbm, out_hbm, idx_v, rows_v, sem):
    wid = lax.axis_index("s") * _NC + lax.axis_index("c")
    row0 = wid * (_EPW // _CHUNK)
    pltpu.sync_copy(idx_hbm.at[pl.ds(row0, _EPW // _CHUNK)], idx_v)

    def body(ci, carry):
        pltpu.async_copy(t_hbm.at[idx_v.at[ci]], rows_v, sem).wait()
        pltpu.sync_copy(
            rows_v, out_hbm.at[pl.ds((row0 + ci) * _CHUNK, _CHUNK)])
        return carry

    lax.fori_loop(0, _EPW // _CHUNK, body, 0)


def _gather_call(t_flat, idx2d):
    mesh = plsc.VectorSubcoreMesh(
        core_axis_name="c", subcore_axis_name="s",
        num_cores=_NC, num_subcores=_NS)
    f = pl.kernel(
        _gather_body,
        out_type=jax.ShapeDtypeStruct((_E, _TPAD), jnp.float32),
        mesh=mesh,
        compiler_params=pltpu.CompilerParams(use_tc_tiling_on_sc=False),
        scratch_types=[
            pltpu.VMEM((_EPW // _CHUNK, _CHUNK), jnp.int32),
            pltpu.VMEM((_CHUNK, _TPAD), jnp.float32),
            pltpu.SemaphoreType.DMA,
        ],
    )
    return f(t_flat, idx2d)


def _mlp_body(t_ref, g_ref, wi_ref, wg_ref, b1_ref, w2_ref, b2_ref,
              w3_ref, b3_ref, wl_ref, bl_ref, out_ref):
    t = t_ref[0]
    bf = jnp.bfloat16
    wg = wg_ref[...].astype(bf)
    w2 = w2_ref[...].astype(bf)
    w3 = w3_ref[...].astype(bf)
    a_i = jnp.dot(t.astype(bf), wi_ref[...].astype(bf),
                  preferred_element_type=jnp.float32)
    a_i = a_i + b1_ref[...]
    sub = 4
    kc = _K // sub
    rows = kc * _RC
    agg = None
    for s in range(sub):
        gs = g_ref[0, s * kc:(s + 1) * kc, :, :]
        gs2 = gs.reshape(rows, _TPAD)
        a_j = jnp.dot(gs2.astype(bf), wg, preferred_element_type=jnp.float32)
        a_i_rep = jnp.broadcast_to(
            a_i[None, :, :], (kc, _RC, _H)).reshape(rows, _H)
        h1 = jnp.maximum(a_j + a_i_rep, 0.0)
        h2 = jnp.dot(h1.astype(bf), w2, preferred_element_type=jnp.float32)
        h2 = jnp.maximum(h2 + b2_ref[...], 0.0)
        kfull = jnp.dot(h2.astype(bf), w3, preferred_element_type=jnp.float32)
        kw = (kfull[:, 0:1] + b3_ref[...]) * (1.0 / math.sqrt(_K))
        msg = kw * gs2[:, :_D]
        agg_s = jnp.sum(msg.reshape(kc, _RC, _D), axis=0)
        agg = agg_s if agg is None else agg + agg_s
    u_blk = t[:, :_D]
    out = jnp.dot(u_blk, wl_ref[...], preferred_element_type=jnp.float32)
    out = out + bl_ref[...] + agg + u_blk
    out_ref[0] = jnp.maximum(out, 0.0)


def _mlp_call(t3, g3, wi, wg, b1, w2, b2, w3r, b3, wl, bl):
    full = lambda s: pl.BlockSpec(s, lambda b, i: tuple(0 for _ in s))
    return pl.pallas_call(
        _mlp_body,
        grid=(_B, _N // _RC),
        in_specs=[
            pl.BlockSpec((1, _RC, _TPAD), lambda b, i: (b, i, 0)),
            pl.BlockSpec((1, _K, _RC, _TPAD), lambda b, i: (b, 0, i, 0)),
            full((_TPAD, _H)),
            full((_TPAD, _H)),
            full((1, _H)),
            full((_H, _H)),
            full((1, _H)),
            full((_H, _D)),
            full((1, 1)),
            full((_D, _D)),
            full((1, _D)),
        ],
        out_specs=pl.BlockSpec((1, _RC, _D), lambda b, i: (b, i, 0)),
        out_shape=jax.ShapeDtypeStruct((_B, _N, _D), jnp.float32),
    )(t3, g3, wi, wg, b1, w2, b2, w3r, b3, wl, bl)


def kernel(u, coords, W_local, b_local, W1, b1, W2, b2, W3, b3):
    b, n, d = u.shape
    c = coords.shape[-1]
    pad = jnp.zeros((b, n, _TPAD - d - c), jnp.float32)
    t3 = jnp.concatenate([u, coords, pad], axis=-1)
    coordst = jnp.swapaxes(coords, 1, 2)

    idx = _knn_call(coords, coordst)
    idx2d = idx.reshape(_E // _CHUNK, _CHUNK)

    t_flat = t3.reshape(b * n, _TPAD)
    g = _gather_call(t_flat, idx2d)
    g4 = g.reshape(b, _K, n, _TPAD)

    zpad = jnp.zeros((_TPAD - d - c, _H), jnp.float32)
    wi = jnp.concatenate([W1[2 * c:2 * c + d], W1[0:c], zpad], axis=0)
    wg = jnp.concatenate([W1[2 * c + d:], W1[c:2 * c], zpad], axis=0)
    w3p = jnp.concatenate([W3, jnp.zeros((_H, _D - 1), jnp.float32)], axis=1)

    out = _mlp_call(
        t3, g4, wi, wg,
        b1.reshape(1, _H), W2, b2.reshape(1, _H),
        w3p, b3.reshape(1, 1),
        W_local, b_local.reshape(1, _D))
    return out


# speedup vs baseline: 13.5961x; 1.2082x over previous
import functools
import math

import jax
import jax.numpy as jnp
from jax import lax
from jax.experimental import pallas as pl
from jax.experimental.pallas import tpu as pltpu
from jax.experimental.pallas import tpu_sc as plsc

_B, _N, _D, _C, _H, _K = 2, 2048, 128, 3, 256, 16
_TPAD = 256
_RA = 256
_RC = 256

_NC, _NS = 2, 16
_NW = _NC * _NS
_E = _B * _N * _K
_EPW = _E // _NW
_CHUNK = 128


def _knn_body(coords_ref, coordst_ref, idx_ref):
    b = pl.program_id(0)
    xi = coords_ref[0]
    xt = coordst_ref[0]
    d0 = xi[:, 0:1] - xt[0:1, :]
    acc = d0 * d0
    d1 = xi[:, 1:2] - xt[1:2, :]
    acc = acc + d1 * d1
    d2 = xi[:, 2:3] - xt[2:3, :]
    acc = acc + d2 * d2

    iota = lax.broadcasted_iota(jnp.int32, (_RA, _N), 1).astype(jnp.float32)
    big = jnp.float32(_N)
    cols = []
    dist = acc
    for _ in range(_K):
        m = jnp.min(dist, axis=1, keepdims=True)
        cand = jnp.where(dist == m, iota, big)
        j = jnp.min(cand, axis=1, keepdims=True)
        cols.append(j)
        dist = jnp.where(cand == j, jnp.float32(jnp.inf), dist)
    idx = jnp.concatenate(cols, axis=1).astype(jnp.int32)
    idx_ref[0] = idx.T + b * _N


def _knn_call(coords, coordst):
    return pl.pallas_call(
        _knn_body,
        grid=(_B, _N // _RA),
        in_specs=[
            pl.BlockSpec((1, _RA, _C), lambda b, i: (b, i, 0)),
            pl.BlockSpec((1, _C, _N), lambda b, i: (b, 0, 0)),
        ],
        out_specs=pl.BlockSpec((1, _K, _RA), lambda b, i: (b, 0, i)),
        out_shape=jax.ShapeDtypeStruct((_B, _K, _N), jnp.int32),
    )(coords, coordst)


def _gather_body(t_hbm, idx_hbm, out_hbm, idx_v, rows_v, sem):
    wid = lax.axis_index("s") * _NC + lax.axis_index("c")
    row0 = wid * (_EPW // _CHUNK)
    pltpu.sync_copy(idx_hbm.at[pl.ds(row0, _EPW // _CHUNK)], idx_v)

    def body(ci, carry):
        pltpu.async_copy(t_hbm.at[idx_v.at[ci]], rows_v, sem).wait()
        pltpu.sync_copy(
            rows_v, out_hbm.at[pl.ds((row0 + ci) * _CHUNK, _CHUNK)])
        return carry

    lax.fori_loop(0, _EPW // _CHUNK, body, 0)


def _gather_call(t_flat, idx2d):
    mesh = plsc.VectorSubcoreMesh(
        core_axis_name="c", subcore_axis_name="s",
        num_cores=_NC, num_subcores=_NS)
    f = pl.kernel(
        _gather_body,
        out_type=jax.ShapeDtypeStruct((_E, _TPAD), jnp.float32),
        mesh=mesh,
        scratch_types=[
            pltpu.VMEM((_EPW // _CHUNK, _CHUNK), jnp.int32),
            pltpu.VMEM((_CHUNK, _TPAD), jnp.float32),
            pltpu.SemaphoreType.DMA,
        ],
    )
    return f(t_flat, idx2d)


def _mlp_body(t_ref, g_ref, wi_ref, wg_ref, b1_ref, w2_ref, b2_ref,
              w3_ref, b3_ref, wl_ref, bl_ref, out_ref):
    t = t_ref[0]
    bf = jnp.bfloat16
    wg = wg_ref[...].astype(bf)
    w2 = w2_ref[...].astype(bf)
    w3 = w3_ref[...].astype(bf)
    a_i = jnp.dot(t.astype(bf), wi_ref[...].astype(bf),
                  preferred_element_type=jnp.float32)
    a_i = a_i + b1_ref[...]
    sub = 4
    kc = _K // sub
    rows = kc * _RC
    agg = None
    for s in range(sub):
        gs = g_ref[0, s * kc:(s + 1) * kc, :, :]
        gs2 = gs.reshape(rows, _TPAD)
        a_j = jnp.dot(gs2.astype(bf), wg, preferred_element_type=jnp.float32)
        a_i_rep = jnp.broadcast_to(
            a_i[None, :, :], (kc, _RC, _H)).reshape(rows, _H)
        h1 = jnp.maximum(a_j + a_i_rep, 0.0)
        h2 = jnp.dot(h1.astype(bf), w2, preferred_element_type=jnp.float32)
        h2 = jnp.maximum(h2 + b2_ref[...], 0.0)
        kfull = jnp.dot(h2.astype(bf), w3, preferred_element_type=jnp.float32)
        kw = (kfull[:, 0:1] + b3_ref[...]) * (1.0 / math.sqrt(_K))
        msg = kw * gs2[:, :_D]
        agg_s = jnp.sum(msg.reshape(kc, _RC, _D), axis=0)
        agg = agg_s if agg is None else agg + agg_s
    u_blk = t[:, :_D]
    out = jnp.dot(u_blk, wl_ref[...], preferred_element_type=jnp.float32)
    out = out + bl_ref[...] + agg + u_blk
    out_ref[0] = jnp.maximum(out, 0.0)


def _mlp_call(t3, g3, wi, wg, b1, w2, b2, w3r, b3, wl, bl):
    full = lambda s: pl.BlockSpec(s, lambda b, i: tuple(0 for _ in s))
    return pl.pallas_call(
        _mlp_body,
        grid=(_B, _N // _RC),
        in_specs=[
            pl.BlockSpec((1, _RC, _TPAD), lambda b, i: (b, i, 0)),
            pl.BlockSpec((1, _K, _RC, _TPAD), lambda b, i: (b, 0, i, 0)),
            full((_TPAD, _H)),
            full((_TPAD, _H)),
            full((1, _H)),
            full((_H, _H)),
            full((1, _H)),
            full((_H, _D)),
            full((1, 1)),
            full((_D, _D)),
            full((1, _D)),
        ],
        out_specs=pl.BlockSpec((1, _RC, _D), lambda b, i: (b, i, 0)),
        out_shape=jax.ShapeDtypeStruct((_B, _N, _D), jnp.float32),
    )(t3, g3, wi, wg, b1, w2, b2, w3r, b3, wl, bl)


def kernel(u, coords, W_local, b_local, W1, b1, W2, b2, W3, b3):
    b, n, d = u.shape
    c = coords.shape[-1]
    pad = jnp.zeros((b, n, _TPAD - d - c), jnp.float32)
    t3 = jnp.concatenate([u, coords, pad], axis=-1)
    coordst = jnp.swapaxes(coords, 1, 2)

    idx = _knn_call(coords, coordst)
    idx2d = idx.reshape(_E // _CHUNK, _CHUNK)

    t_flat = t3.reshape(b * n, _TPAD)
    g = _gather_call(t_flat, idx2d)
    g4 = g.reshape(b, _K, n, _TPAD)

    zpad = jnp.zeros((_TPAD - d - c, _H), jnp.float32)
    wi = jnp.concatenate([W1[2 * c:2 * c + d], W1[0:c], zpad], axis=0)
    wg = jnp.concatenate([W1[2 * c + d:], W1[c:2 * c], zpad], axis=0)
    w3p = jnp.concatenate([W3, jnp.zeros((_H, _D - 1), jnp.float32)], axis=1)

    out = _mlp_call(
        t3, g4, wi, wg,
        b1.reshape(1, _H), W2, b2.reshape(1, _H),
        w3p, b3.reshape(1, 1),
        W_local, b_local.reshape(1, _D))
    return out


# speedup vs baseline: 15.1725x; 1.1159x over previous
import functools
import math

import jax
import jax.numpy as jnp
from jax import lax
from jax.experimental import pallas as pl
from jax.experimental.pallas import tpu as pltpu
from jax.experimental.pallas import tpu_sc as plsc

_B, _N, _D, _C, _H, _K = 2, 2048, 128, 3, 256, 16
_TPAD = 256
_RA = 256
_RC = 256

_NC, _NS = 2, 16
_NW = _NC * _NS
_E = _B * _N * _K
_EPW = _E // _NW
_CHUNK = 128


def _knn_body(coords_ref, coordst_ref, idx_ref):
    b = pl.program_id(0)
    xi = coords_ref[0]
    xt = coordst_ref[0]
    d0 = xi[:, 0:1] - xt[0:1, :]
    acc = d0 * d0
    d1 = xi[:, 1:2] - xt[1:2, :]
    acc = acc + d1 * d1
    d2 = xi[:, 2:3] - xt[2:3, :]
    acc = acc + d2 * d2

    half = _N // 2
    d_a = acc[:, :half]
    d_b = acc[:, half:]
    iotah = lax.broadcasted_iota(jnp.int32, (_RA, half), 1).astype(jnp.float32)
    cond = d_b < d_a
    lo = jnp.minimum(d_a, d_b)
    hi = jnp.maximum(d_a, d_b)
    elo = jnp.where(cond, iotah + half, iotah)
    ehi = jnp.where(cond, iotah, iotah + half)
    big = jnp.float32(_N)
    inf = jnp.float32(jnp.inf)
    cols = []
    for _ in range(_K):
        m = jnp.min(lo, axis=1, keepdims=True)
        j = jnp.min(jnp.where(lo == m, elo, big), axis=1, keepdims=True)
        cols.append(j)
        pick = elo == j
        lo = jnp.where(pick, hi, lo)
        elo = jnp.where(pick, ehi, elo)
        hi = jnp.where(pick, inf, hi)
    idx = jnp.concatenate(cols, axis=1).astype(jnp.int32)
    idx_ref[0] = idx.T + b * _N


def _knn_call(coords, coordst):
    return pl.pallas_call(
        _knn_body,
        grid=(_B, _N // _RA),
        in_specs=[
            pl.BlockSpec((1, _RA, _C), lambda b, i: (b, i, 0)),
            pl.BlockSpec((1, _C, _N), lambda b, i: (b, 0, 0)),
        ],
        out_specs=pl.BlockSpec((1, _K, _RA), lambda b, i: (b, 0, i)),
        out_shape=jax.ShapeDtypeStruct((_B, _K, _N), jnp.int32),
    )(coords, coordst)


def _gather_body(t_hbm, idx_hbm, out_hbm, idx_v, rv0, rv1, sem0, sem1):
    wid = lax.axis_index("s") * _NC + lax.axis_index("c")
    nchunk = _EPW // _CHUNK
    row0 = wid * nchunk
    pltpu.sync_copy(idx_hbm.at[pl.ds(row0, nchunk)], idx_v)

    rv = (rv0, rv1)
    sems = (sem0, sem1)
    prev = None
    for ci in range(nchunk):
        cur = pltpu.async_copy(
            t_hbm.at[idx_v.at[ci]], rv[ci % 2], sems[ci % 2])
        if prev is not None:
            prev.wait()
            pltpu.sync_copy(
                rv[(ci - 1) % 2],
                out_hbm.at[pl.ds((row0 + ci - 1) * _CHUNK, _CHUNK)])
        prev = cur
    prev.wait()
    pltpu.sync_copy(
        rv[(nchunk - 1) % 2],
        out_hbm.at[pl.ds((row0 + nchunk - 1) * _CHUNK, _CHUNK)])


def _gather_call(t_flat, idx2d):
    mesh = plsc.VectorSubcoreMesh(
        core_axis_name="c", subcore_axis_name="s",
        num_cores=_NC, num_subcores=_NS)
    f = pl.kernel(
        _gather_body,
        out_type=jax.ShapeDtypeStruct((_E, _D), jnp.int32),
        mesh=mesh,
        scratch_types=[
            pltpu.VMEM((_EPW // _CHUNK, _CHUNK), jnp.int32),
            pltpu.VMEM((_CHUNK, _D), jnp.int32),
            pltpu.VMEM((_CHUNK, _D), jnp.int32),
            pltpu.SemaphoreType.DMA,
            pltpu.SemaphoreType.DMA,
        ],
    )
    return f(t_flat, idx2d)


def _mlp_body(t_ref, g_ref, wi_ref, wgu_ref, wgx_ref, b1_ref, w2_ref, b2_ref,
              w3_ref, b3_ref, wl_ref, bl_ref, out_ref):
    t = t_ref[0]
    bf = jnp.bfloat16
    wgu = wgu_ref[...].astype(bf)
    wgx = wgx_ref[...].astype(bf)
    w2 = w2_ref[...].astype(bf)
    w3 = w3_ref[...].astype(bf)
    a_i = jnp.dot(t.astype(bf), wi_ref[...].astype(bf),
                  preferred_element_type=jnp.float32)
    a_i = a_i + b1_ref[...]
    sub = 4
    kc = _K // sub
    rows = kc * _RC
    agg = None
    for s in range(sub):
        gs = g_ref[0, s * kc:(s + 1) * kc, :, :].reshape(rows, _D)
        u_f = lax.bitcast_convert_type(gs << 16, jnp.float32)
        x_f = lax.bitcast_convert_type(gs & jnp.int32(-65536), jnp.float32)
        a_j = (jnp.dot(u_f.astype(bf), wgu, preferred_element_type=jnp.float32)
               + jnp.dot(x_f.astype(bf), wgx,
                         preferred_element_type=jnp.float32))
        a_i_rep = jnp.broadcast_to(
            a_i[None, :, :], (kc, _RC, _H)).reshape(rows, _H)
        h1 = jnp.maximum(a_j + a_i_rep, 0.0)
        h2 = jnp.dot(h1.astype(bf), w2, preferred_element_type=jnp.float32)
        h2 = jnp.maximum(h2 + b2_ref[...], 0.0)
        kfull = jnp.dot(h2.astype(bf), w3, preferred_element_type=jnp.float32)
        kw = (kfull[:, 0:1] + b3_ref[...]) * (1.0 / math.sqrt(_K))
        msg = kw * u_f
        agg_s = jnp.sum(msg.reshape(kc, _RC, _D), axis=0)
        agg = agg_s if agg is None else agg + agg_s
    u_blk = t[:, :_D]
    out = jnp.dot(u_blk, wl_ref[...], preferred_element_type=jnp.float32)
    out = out + bl_ref[...] + agg + u_blk
    out_ref[0] = jnp.maximum(out, 0.0)


def _mlp_call(t3, g4, wi, wgu, wgx, b1, w2, b2, w3r, b3, wl, bl):
    full = lambda s: pl.BlockSpec(s, lambda b, i: tuple(0 for _ in s))
    return pl.pallas_call(
        _mlp_body,
        grid=(_B, _N // _RC),
        in_specs=[
            pl.BlockSpec((1, _RC, _TPAD), lambda b, i: (b, i, 0)),
            pl.BlockSpec((1, _K, _RC, _D), lambda b, i: (b, 0, i, 0)),
            full((_TPAD, _H)),
            full((_D, _H)),
            full((_D, _H)),
            full((1, _H)),
            full((_H, _H)),
            full((1, _H)),
            full((_H, _D)),
            full((1, 1)),
            full((_D, _D)),
            full((1, _D)),
        ],
        out_specs=pl.BlockSpec((1, _RC, _D), lambda b, i: (b, i, 0)),
        out_shape=jax.ShapeDtypeStruct((_B, _N, _D), jnp.float32),
    )(t3, g4, wi, wgu, wgx, b1, w2, b2, w3r, b3, wl, bl)


def kernel(u, coords, W_local, b_local, W1, b1, W2, b2, W3, b3):
    b, n, d = u.shape
    c = coords.shape[-1]
    pad = jnp.zeros((b, n, _TPAD - d - c), jnp.float32)
    t3 = jnp.concatenate([u, coords, pad], axis=-1)
    coordst = jnp.swapaxes(coords, 1, 2)

    idx = _knn_call(coords, coordst)
    idx2d = idx.reshape(_E // _CHUNK, _CHUNK)

    ub = lax.bitcast_convert_type(u.astype(jnp.bfloat16), jnp.uint16)
    xb = lax.bitcast_convert_type(
        jnp.concatenate([coords, jnp.zeros((b, n, _D - c), jnp.float32)],
                        axis=-1).astype(jnp.bfloat16), jnp.uint16)
    tpack = lax.bitcast_convert_type(
        ub.astype(jnp.uint32) | (xb.astype(jnp.uint32) << 16), jnp.int32)
    t_flat = tpack.reshape(b * n, _D)
    g = _gather_call(t_flat, idx2d)
    g4 = g.reshape(b, _K, n, _D)

    zpad = jnp.zeros((_TPAD - d - c, _H), jnp.float32)
    wi = jnp.concatenate([W1[2 * c:2 * c + d], W1[0:c], zpad], axis=0)
    wgu = W1[2 * c + d:]
    wgx = jnp.concatenate(
        [W1[c:2 * c], jnp.zeros((_D - c, _H), jnp.float32)], axis=0)
    w3p = jnp.concatenate([W3, jnp.zeros((_H, _D - 1), jnp.float32)], axis=1)

    out = _mlp_call(
        t3, g4, wi, wgu, wgx,
        b1.reshape(1, _H), W2, b2.reshape(1, _H),
        w3p, b3.reshape(1, 1),
        W_local, b_local.reshape(1, _D))
    return out


# speedup vs baseline: 15.5104x; 1.0223x over previous
import functools
import math

import jax
import jax.numpy as jnp
from jax import lax
from jax.experimental import pallas as pl
from jax.experimental.pallas import tpu as pltpu
from jax.experimental.pallas import tpu_sc as plsc

_B, _N, _D, _C, _H, _K = 2, 2048, 128, 3, 256, 16
_TPAD = 256
_RA = 256
_RC = 256

_NC, _NS = 2, 16
_NW = _NC * _NS
_E = _B * _N * _K
_EPW = _E // _NW
_CHUNK = 128


def _knn_body(u_ref, coords_ref, coordst_ref, idx_ref, t3_ref, tp_ref):
    b = pl.program_id(0)
    xi = coords_ref[0]
    xt = coordst_ref[0]

    ub = u_ref[0]
    xpad = jnp.concatenate(
        [xi, jnp.zeros((_RA, _D - _C), jnp.float32)], axis=1)
    t3_ref[0] = jnp.concatenate([ub, xpad], axis=1)
    ub16 = lax.bitcast_convert_type(ub.astype(jnp.bfloat16), jnp.uint16)
    xb16 = lax.bitcast_convert_type(xpad.astype(jnp.bfloat16), jnp.uint16)
    tp_ref[0] = lax.bitcast_convert_type(
        ub16.astype(jnp.uint32) | (xb16.astype(jnp.uint32) << 16), jnp.int32)
    d0 = xi[:, 0:1] - xt[0:1, :]
    acc = d0 * d0
    d1 = xi[:, 1:2] - xt[1:2, :]
    acc = acc + d1 * d1
    d2 = xi[:, 2:3] - xt[2:3, :]
    acc = acc + d2 * d2

    half = _N // 2
    d_a = acc[:, :half]
    d_b = acc[:, half:]
    iotah = lax.broadcasted_iota(jnp.int32, (_RA, half), 1).astype(jnp.float32)
    cond = d_b < d_a
    lo = jnp.minimum(d_a, d_b)
    hi = jnp.maximum(d_a, d_b)
    elo = jnp.where(cond, iotah + half, iotah)
    ehi = jnp.where(cond, iotah, iotah + half)
    big = jnp.float32(_N)
    inf = jnp.float32(jnp.inf)
    cols = []
    for _ in range(_K):
        m = jnp.min(lo, axis=1, keepdims=True)
        j = jnp.min(jnp.where(lo == m, elo, big), axis=1, keepdims=True)
        cols.append(j)
        pick = elo == j
        lo = jnp.where(pick, hi, lo)
        elo = jnp.where(pick, ehi, elo)
        hi = jnp.where(pick, inf, hi)
    idx = jnp.concatenate(cols, axis=1).astype(jnp.int32)
    idx_ref[0] = idx.T + b * _N


def _knn_call(u, coords, coordst):
    return pl.pallas_call(
        _knn_body,
        grid=(_B, _N // _RA),
        in_specs=[
            pl.BlockSpec((1, _RA, _D), lambda b, i: (b, i, 0)),
            pl.BlockSpec((1, _RA, _C), lambda b, i: (b, i, 0)),
            pl.BlockSpec((1, _C, _N), lambda b, i: (b, 0, 0)),
        ],
        out_specs=[
            pl.BlockSpec((1, _K, _RA), lambda b, i: (b, 0, i)),
            pl.BlockSpec((1, _RA, _TPAD), lambda b, i: (b, i, 0)),
            pl.BlockSpec((1, _RA, _D), lambda b, i: (b, i, 0)),
        ],
        out_shape=[
            jax.ShapeDtypeStruct((_B, _K, _N), jnp.int32),
            jax.ShapeDtypeStruct((_B, _N, _TPAD), jnp.float32),
            jax.ShapeDtypeStruct((_B, _N, _D), jnp.int32),
        ],
    )(u, coords, coordst)


def _gather_body(t_hbm, idx_hbm, out_hbm, idx_v, rv0, rv1, sem0, sem1):
    wid = lax.axis_index("s") * _NC + lax.axis_index("c")
    nchunk = _EPW // _CHUNK
    row0 = wid * nchunk
    pltpu.sync_copy(idx_hbm.at[pl.ds(row0, nchunk)], idx_v)

    rv = (rv0, rv1)
    sems = (sem0, sem1)
    prev = None
    for ci in range(nchunk):
        cur = pltpu.async_copy(
            t_hbm.at[idx_v.at[ci]], rv[ci % 2], sems[ci % 2])
        if prev is not None:
            prev.wait()
            pltpu.sync_copy(
                rv[(ci - 1) % 2],
                out_hbm.at[pl.ds((row0 + ci - 1) * _CHUNK, _CHUNK)])
        prev = cur
    prev.wait()
    pltpu.sync_copy(
        rv[(nchunk - 1) % 2],
        out_hbm.at[pl.ds((row0 + nchunk - 1) * _CHUNK, _CHUNK)])


def _gather_call(t_flat, idx2d):
    mesh = plsc.VectorSubcoreMesh(
        core_axis_name="c", subcore_axis_name="s",
        num_cores=_NC, num_subcores=_NS)
    f = pl.kernel(
        _gather_body,
        out_type=jax.ShapeDtypeStruct((_E, _D), jnp.int32),
        mesh=mesh,
        scratch_types=[
            pltpu.VMEM((_EPW // _CHUNK, _CHUNK), jnp.int32),
            pltpu.VMEM((_CHUNK, _D), jnp.int32),
            pltpu.VMEM((_CHUNK, _D), jnp.int32),
            pltpu.SemaphoreType.DMA,
            pltpu.SemaphoreType.DMA,
        ],
    )
    return f(t_flat, idx2d)


def _mlp_body(t_ref, g_ref, wi_ref, wgu_ref, wgx_ref, b1_ref, w2_ref, b2_ref,
              w3_ref, b3_ref, wl_ref, bl_ref, out_ref):
    t = t_ref[0]
    bf = jnp.bfloat16
    wgu = wgu_ref[...].astype(bf)
    wgx = wgx_ref[...].astype(bf)
    w2 = w2_ref[...].astype(bf)
    w3 = w3_ref[...].astype(bf)
    a_i = jnp.dot(t.astype(bf), wi_ref[...].astype(bf),
                  preferred_element_type=jnp.float32)
    a_i = a_i + b1_ref[...]
    sub = 4
    kc = _K // sub
    rows = kc * _RC
    agg = None
    for s in range(sub):
        gs = g_ref[0, s * kc:(s + 1) * kc, :, :].reshape(rows, _D)
        u_f = lax.bitcast_convert_type(gs << 16, jnp.float32)
        x_f = lax.bitcast_convert_type(gs & jnp.int32(-65536), jnp.float32)
        a_j = (jnp.dot(u_f.astype(bf), wgu, preferred_element_type=jnp.float32)
               + jnp.dot(x_f.astype(bf), wgx,
                         preferred_element_type=jnp.float32))
        a_i_rep = jnp.broadcast_to(
            a_i[None, :, :], (kc, _RC, _H)).reshape(rows, _H)
        h1 = jnp.maximum(a_j + a_i_rep, 0.0)
        h2 = jnp.dot(h1.astype(bf), w2, preferred_element_type=jnp.float32)
        h2 = jnp.maximum(h2 + b2_ref[...], 0.0)
        kfull = jnp.dot(h2.astype(bf), w3, preferred_element_type=jnp.float32)
        kw = (kfull[:, 0:1] + b3_ref[...]) * (1.0 / math.sqrt(_K))
        msg = kw * u_f
        agg_s = jnp.sum(msg.reshape(kc, _RC, _D), axis=0)
        agg = agg_s if agg is None else agg + agg_s
    u_blk = t[:, :_D]
    out = jnp.dot(u_blk, wl_ref[...], preferred_element_type=jnp.float32)
    out = out + bl_ref[...] + agg + u_blk
    out_ref[0] = jnp.maximum(out, 0.0)


def _mlp_call(t3, g4, wi, wgu, wgx, b1, w2, b2, w3r, b3, wl, bl):
    full = lambda s: pl.BlockSpec(s, lambda b, i: tuple(0 for _ in s))
    return pl.pallas_call(
        _mlp_body,
        grid=(_B, _N // _RC),
        in_specs=[
            pl.BlockSpec((1, _RC, _TPAD), lambda b, i: (b, i, 0)),
            pl.BlockSpec((1, _K, _RC, _D), lambda b, i: (b, 0, i, 0)),
            full((_TPAD, _H)),
            full((_D, _H)),
            full((_D, _H)),
            full((1, _H)),
            full((_H, _H)),
            full((1, _H)),
            full((_H, _D)),
            full((1, 1)),
            full((_D, _D)),
            full((1, _D)),
        ],
        out_specs=pl.BlockSpec((1, _RC, _D), lambda b, i: (b, i, 0)),
        out_shape=jax.ShapeDtypeStruct((_B, _N, _D), jnp.float32),
    )(t3, g4, wi, wgu, wgx, b1, w2, b2, w3r, b3, wl, bl)


def kernel(u, coords, W_local, b_local, W1, b1, W2, b2, W3, b3):
    b, n, d = u.shape
    c = coords.shape[-1]
    coordst = jnp.swapaxes(coords, 1, 2)

    idx, t3, tpack = _knn_call(u, coords, coordst)
    idx2d = idx.reshape(_E // _CHUNK, _CHUNK)

    t_flat = tpack.reshape(b * n, _D)
    g = _gather_call(t_flat, idx2d)
    g4 = g.reshape(b, _K, n, _D)

    zpad = jnp.zeros((_TPAD - d - c, _H), jnp.float32)
    wi = jnp.concatenate([W1[2 * c:2 * c + d], W1[0:c], zpad], axis=0)
    wgu = W1[2 * c + d:]
    wgx = jnp.concatenate(
        [W1[c:2 * c], jnp.zeros((_D - c, _H), jnp.float32)], axis=0)
    w3p = jnp.concatenate([W3, jnp.zeros((_H, _D - 1), jnp.float32)], axis=1)

    out = _mlp_call(
        t3, g4, wi, wgu, wgx,
        b1.reshape(1, _H), W2, b2.reshape(1, _H),
        w3p, b3.reshape(1, 1),
        W_local, b_local.reshape(1, _D))
    return out


# speedup vs baseline: 16.1360x; 1.0403x over previous
import functools
import math

import jax
import jax.numpy as jnp
from jax import lax
from jax.experimental import pallas as pl
from jax.experimental.pallas import tpu as pltpu
from jax.experimental.pallas import tpu_sc as plsc

_B, _N, _D, _C, _H, _K = 2, 2048, 128, 3, 256, 16
_TPAD = 256
_RA = 256
_RC = 256

_NC, _NS = 2, 16
_NW = _NC * _NS
_E = _B * _N * _K
_EPW = _E // _NW
_CHUNK = 128


def _knn_body(u_ref, coords_ref, coordst_ref, idx_ref, t3_ref, tp_ref):
    xi = coords_ref[0]
    xt = coordst_ref[0]

    ub = u_ref[0]
    xpad = jnp.concatenate(
        [xi, jnp.zeros((_RA, _D - _C), jnp.float32)], axis=1)
    t3_ref[0] = jnp.concatenate([ub, xpad], axis=1)
    ub16 = lax.bitcast_convert_type(ub.astype(jnp.bfloat16), jnp.uint16)
    xb16 = lax.bitcast_convert_type(xpad.astype(jnp.bfloat16), jnp.uint16)
    tp_ref[0] = lax.bitcast_convert_type(
        ub16.astype(jnp.uint32) | (xb16.astype(jnp.uint32) << 16), jnp.int32)
    d0 = xi[:, 0:1] - xt[0:1, :]
    acc = d0 * d0
    d1 = xi[:, 1:2] - xt[1:2, :]
    acc = acc + d1 * d1
    d2 = xi[:, 2:3] - xt[2:3, :]
    acc = acc + d2 * d2

    half = _N // 2
    d_a = acc[:, :half]
    d_b = acc[:, half:]
    iotah = lax.broadcasted_iota(jnp.int32, (_RA, half), 1).astype(jnp.float32)
    cond = d_b < d_a
    lo = jnp.minimum(d_a, d_b)
    hi = jnp.maximum(d_a, d_b)
    elo = jnp.where(cond, iotah + half, iotah)
    ehi = jnp.where(cond, iotah, iotah + half)
    big = jnp.float32(_N)
    inf = jnp.float32(jnp.inf)
    cols = []
    for _ in range(_K):
        m = jnp.min(lo, axis=1, keepdims=True)
        j = jnp.min(jnp.where(lo == m, elo, big), axis=1, keepdims=True)
        cols.append(j)
        pick = elo == j
        lo = jnp.where(pick, hi, lo)
        elo = jnp.where(pick, ehi, elo)
        hi = jnp.where(pick, inf, hi)
    idx = jnp.concatenate(cols, axis=1).astype(jnp.int32)
    idx_ref[0] = idx.T


def _knn_call(u, coords, coordst):
    return pl.pallas_call(
        _knn_body,
        grid=(_N // _RA,),
        in_specs=[
            pl.BlockSpec((1, _RA, _D), lambda i: (0, i, 0)),
            pl.BlockSpec((1, _RA, _C), lambda i: (0, i, 0)),
            pl.BlockSpec((1, _C, _N), lambda i: (0, 0, 0)),
        ],
        out_specs=[
            pl.BlockSpec((1, _K, _RA), lambda i: (0, 0, i)),
            pl.BlockSpec((1, _RA, _TPAD), lambda i: (0, i, 0)),
            pl.BlockSpec((1, _RA, _D), lambda i: (0, i, 0)),
        ],
        out_shape=[
            jax.ShapeDtypeStruct((1, _K, _N), jnp.int32),
            jax.ShapeDtypeStruct((1, _N, _TPAD), jnp.float32),
            jax.ShapeDtypeStruct((1, _N, _D), jnp.int32),
        ],
    )(u, coords, coordst)


def _gather_body(t_hbm, idx_hbm, out_hbm, idx_v, rv0, rv1, sem0, sem1):
    wid = lax.axis_index("s") * _NC + lax.axis_index("c")
    nchunk = idx_v.shape[0]
    row0 = wid * nchunk
    pltpu.sync_copy(idx_hbm.at[pl.ds(row0, nchunk)], idx_v)

    rv = (rv0, rv1)
    sems = (sem0, sem1)
    prev = None
    for ci in range(nchunk):
        cur = pltpu.async_copy(
            t_hbm.at[idx_v.at[ci]], rv[ci % 2], sems[ci % 2])
        if prev is not None:
            prev.wait()
            pltpu.sync_copy(
                rv[(ci - 1) % 2],
                out_hbm.at[pl.ds((row0 + ci - 1) * _CHUNK, _CHUNK)])
        prev = cur
    prev.wait()
    pltpu.sync_copy(
        rv[(nchunk - 1) % 2],
        out_hbm.at[pl.ds((row0 + nchunk - 1) * _CHUNK, _CHUNK)])


def _gather_call(t_flat, idx2d):
    mesh = plsc.VectorSubcoreMesh(
        core_axis_name="c", subcore_axis_name="s",
        num_cores=_NC, num_subcores=_NS)
    ne = idx2d.shape[0] * _CHUNK
    f = pl.kernel(
        _gather_body,
        out_type=jax.ShapeDtypeStruct((ne, _D), jnp.int32),
        mesh=mesh,
        scratch_types=[
            pltpu.VMEM((ne // _NW // _CHUNK, _CHUNK), jnp.int32),
            pltpu.VMEM((_CHUNK, _D), jnp.int32),
            pltpu.VMEM((_CHUNK, _D), jnp.int32),
            pltpu.SemaphoreType.DMA,
            pltpu.SemaphoreType.DMA,
        ],
    )
    return f(t_flat, idx2d)


def _mlp_body(t_ref, g_ref, wi_ref, wgu_ref, wgx_ref, b1_ref, w2_ref, b2_ref,
              w3_ref, b3_ref, wl_ref, bl_ref, out_ref):
    t = t_ref[0]
    bf = jnp.bfloat16
    wgu = wgu_ref[...].astype(bf)
    wgx = wgx_ref[...].astype(bf)
    w2 = w2_ref[...].astype(bf)
    w3 = w3_ref[...].astype(bf)
    a_i = jnp.dot(t.astype(bf), wi_ref[...].astype(bf),
                  preferred_element_type=jnp.float32)
    a_i = a_i + b1_ref[...]
    sub = 4
    kc = _K // sub
    rows = kc * _RC
    agg = None
    for s in range(sub):
        gs = g_ref[0, s * kc:(s + 1) * kc, :, :].reshape(rows, _D)
        u_f = lax.bitcast_convert_type(gs << 16, jnp.float32)
        x_f = lax.bitcast_convert_type(gs & jnp.int32(-65536), jnp.float32)
        a_j = (jnp.dot(u_f.astype(bf), wgu, preferred_element_type=jnp.float32)
               + jnp.dot(x_f.astype(bf), wgx,
                         preferred_element_type=jnp.float32))
        a_i_rep = jnp.broadcast_to(
            a_i[None, :, :], (kc, _RC, _H)).reshape(rows, _H)
        h1 = jnp.maximum(a_j + a_i_rep, 0.0)
        h2 = jnp.dot(h1.astype(bf), w2, preferred_element_type=jnp.float32)
        h2 = jnp.maximum(h2 + b2_ref[...], 0.0)
        kfull = jnp.dot(h2.astype(bf), w3, preferred_element_type=jnp.float32)
        kw = (kfull[:, 0:1] + b3_ref[...]) * (1.0 / math.sqrt(_K))
        msg = kw * u_f
        agg_s = jnp.sum(msg.reshape(kc, _RC, _D), axis=0)
        agg = agg_s if agg is None else agg + agg_s
    u_blk = t[:, :_D]
    out = jnp.dot(u_blk, wl_ref[...], preferred_element_type=jnp.float32)
    out = out + bl_ref[...] + agg + u_blk
    out_ref[0] = jnp.maximum(out, 0.0)


def _mlp_call(t3, g4, wi, wgu, wgx, b1, w2, b2, w3r, b3, wl, bl):
    full = lambda s: pl.BlockSpec(s, lambda i: tuple(0 for _ in s))
    return pl.pallas_call(
        _mlp_body,
        grid=(_N // _RC,),
        in_specs=[
            pl.BlockSpec((1, _RC, _TPAD), lambda i: (0, i, 0)),
            pl.BlockSpec((1, _K, _RC, _D), lambda i: (0, 0, i, 0)),
            full((_TPAD, _H)),
            full((_D, _H)),
            full((_D, _H)),
            full((1, _H)),
            full((_H, _H)),
            full((1, _H)),
            full((_H, _D)),
            full((1, 1)),
            full((_D, _D)),
            full((1, _D)),
        ],
        out_specs=pl.BlockSpec((1, _RC, _D), lambda i: (0, i, 0)),
        out_shape=jax.ShapeDtypeStruct((1, _N, _D), jnp.float32),
    )(t3, g4, wi, wgu, wgx, b1, w2, b2, w3r, b3, wl, bl)


def kernel(u, coords, W_local, b_local, W1, b1, W2, b2, W3, b3):
    b, n, d = u.shape
    c = coords.shape[-1]
    coordst = jnp.swapaxes(coords, 1, 2)

    zpad = jnp.zeros((_TPAD - d - c, _H), jnp.float32)
    wi = jnp.concatenate([W1[2 * c:2 * c + d], W1[0:c], zpad], axis=0)
    wgu = W1[2 * c + d:]
    wgx = jnp.concatenate(
        [W1[c:2 * c], jnp.zeros((_D - c, _H), jnp.float32)], axis=0)
    w3p = jnp.concatenate([W3, jnp.zeros((_H, _D - 1), jnp.float32)], axis=1)

    outs = []
    for bi in range(b):
        idx, t3, tpack = _knn_call(
            u[bi:bi + 1], coords[bi:bi + 1], coordst[bi:bi + 1])
        idx2d = idx.reshape(n * _K // _CHUNK, _CHUNK)
        g = _gather_call(tpack.reshape(n, _D), idx2d)
        g4 = g.reshape(1, _K, n, _D)
        outs.append(_mlp_call(
            t3, g4, wi, wgu, wgx,
            b1.reshape(1, _H), W2, b2.reshape(1, _H),
            w3p, b3.reshape(1, 1),
            W_local, b_local.reshape(1, _D)))
    return jnp.concatenate(outs, axis=0)


# speedup vs baseline: 16.3325x; 1.0122x over previous
import functools
import math

import jax
import jax.numpy as jnp
from jax import lax
from jax.experimental import pallas as pl
from jax.experimental.pallas import tpu as pltpu
from jax.experimental.pallas import tpu_sc as plsc

_B, _N, _D, _C, _H, _K = 2, 2048, 128, 3, 256, 16
_TPAD = 256
_RA = 256
_RC = 256

_NC, _NS = 2, 16
_NW = _NC * _NS
_E = _B * _N * _K
_EPW = _E // _NW
_CHUNK = 128


def _knn_body(u_ref, coords_ref, coordst_ref, idx_ref, t3_ref, tp_ref):
    xi = coords_ref[0]
    xt = coordst_ref[0]

    ub = u_ref[0]
    xpad = jnp.concatenate(
        [xi, jnp.zeros((_RA, _D - _C), jnp.float32)], axis=1)
    t3_ref[0] = jnp.concatenate([ub, xpad], axis=1)
    ub16 = lax.bitcast_convert_type(ub.astype(jnp.bfloat16), jnp.uint16)
    xb16 = lax.bitcast_convert_type(xpad.astype(jnp.bfloat16), jnp.uint16)
    tp_ref[0] = lax.bitcast_convert_type(
        ub16.astype(jnp.uint32) | (xb16.astype(jnp.uint32) << 16), jnp.int32)
    d0 = xi[:, 0:1] - xt[0:1, :]
    acc = d0 * d0
    d1 = xi[:, 1:2] - xt[1:2, :]
    acc = acc + d1 * d1
    d2 = xi[:, 2:3] - xt[2:3, :]
    acc = acc + d2 * d2

    half = _N // 2
    d_a = acc[:, :half]
    d_b = acc[:, half:]
    iotah = lax.broadcasted_iota(jnp.int32, (_RA, half), 1).astype(jnp.float32)
    cond = d_b < d_a
    lo = jnp.minimum(d_a, d_b)
    hi = jnp.maximum(d_a, d_b)
    elo = jnp.where(cond, iotah + half, iotah)
    ehi = jnp.where(cond, iotah, iotah + half)
    big = jnp.float32(_N)
    inf = jnp.float32(jnp.inf)
    cols = []
    for _ in range(_K):
        m = jnp.min(lo, axis=1, keepdims=True)
        j = jnp.min(jnp.where(lo == m, elo, big), axis=1, keepdims=True)
        cols.append(j)
        pick = elo == j
        lo = jnp.where(pick, hi, lo)
        elo = jnp.where(pick, ehi, elo)
        hi = jnp.where(pick, inf, hi)
    idx = jnp.concatenate(cols, axis=1).astype(jnp.int32)
    idx_ref[0] = idx.T


def _knn_call(u, coords, coordst):
    return pl.pallas_call(
        _knn_body,
        grid=(_N // _RA,),
        in_specs=[
            pl.BlockSpec((1, _RA, _D), lambda i: (0, i, 0)),
            pl.BlockSpec((1, _RA, _C), lambda i: (0, i, 0)),
            pl.BlockSpec((1, _C, _N), lambda i: (0, 0, 0)),
        ],
        out_specs=[
            pl.BlockSpec((1, _K, _RA), lambda i: (0, 0, i)),
            pl.BlockSpec((1, _RA, _TPAD), lambda i: (0, i, 0)),
            pl.BlockSpec((1, _RA, _D), lambda i: (0, i, 0)),
        ],
        out_shape=[
            jax.ShapeDtypeStruct((1, _K, _N), jnp.int32),
            jax.ShapeDtypeStruct((1, _N, _TPAD), jnp.float32),
            jax.ShapeDtypeStruct((1, _N, _D), jnp.int32),
        ],
    )(u, coords, coordst)


def _gather_body(t_hbm, idx_hbm, out_hbm, idx_v, rv0, rv1, sem0, sem1):
    wid = lax.axis_index("s") * _NC + lax.axis_index("c")
    nchunk = idx_v.shape[0]
    row0 = wid * nchunk
    pltpu.sync_copy(idx_hbm.at[pl.ds(row0, nchunk)], idx_v)

    rv = (rv0, rv1)
    sems = (sem0, sem1)
    prev = None
    for ci in range(nchunk):
        cur = pltpu.async_copy(
            t_hbm.at[idx_v.at[ci]], rv[ci % 2], sems[ci % 2])
        if prev is not None:
            prev.wait()
            pltpu.sync_copy(
                rv[(ci - 1) % 2],
                out_hbm.at[pl.ds((row0 + ci - 1) * _CHUNK, _CHUNK)])
        prev = cur
    prev.wait()
    pltpu.sync_copy(
        rv[(nchunk - 1) % 2],
        out_hbm.at[pl.ds((row0 + nchunk - 1) * _CHUNK, _CHUNK)])


def _gather_call(t_flat, idx2d):
    mesh = plsc.VectorSubcoreMesh(
        core_axis_name="c", subcore_axis_name="s",
        num_cores=_NC, num_subcores=_NS)
    ne = idx2d.shape[0] * _CHUNK
    f = pl.kernel(
        _gather_body,
        out_type=jax.ShapeDtypeStruct((ne, _D), jnp.int32),
        mesh=mesh,
        scratch_types=[
            pltpu.VMEM((ne // _NW // _CHUNK, _CHUNK), jnp.int32),
            pltpu.VMEM((_CHUNK, _D), jnp.int32),
            pltpu.VMEM((_CHUNK, _D), jnp.int32),
            pltpu.SemaphoreType.DMA,
            pltpu.SemaphoreType.DMA,
        ],
    )
    return f(t_flat, idx2d)


def _mlp_body(t_ref, g_ref, wi_ref, wgu_ref, wgx_ref, b1_ref, w2_ref, b2_ref,
              w3_ref, b3_ref, wl_ref, bl_ref, out_ref):
    t = t_ref[0]
    bf = jnp.bfloat16
    wgu = wgu_ref[...].astype(bf)
    wgx = wgx_ref[...].astype(bf)
    w2 = w2_ref[...].astype(bf)
    w3 = w3_ref[...].astype(bf)
    a_i = jnp.dot(t.astype(bf), wi_ref[...].astype(bf),
                  preferred_element_type=jnp.float32)
    a_i = a_i + b1_ref[...]
    sub = 8
    kc = _K // sub
    rows = kc * _RC
    agg = None
    for s in range(sub):
        gs = g_ref[0, s * kc:(s + 1) * kc, :, :].reshape(rows, _D)
        u_f = lax.bitcast_convert_type(gs << 16, jnp.float32)
        x_f = lax.bitcast_convert_type(gs & jnp.int32(-65536), jnp.float32)
        a_j = (jnp.dot(u_f.astype(bf), wgu, preferred_element_type=jnp.float32)
               + jnp.dot(x_f.astype(bf), wgx,
                         preferred_element_type=jnp.float32))
        a_i_rep = jnp.broadcast_to(
            a_i[None, :, :], (kc, _RC, _H)).reshape(rows, _H)
        h1 = jnp.maximum(a_j + a_i_rep, 0.0)
        h2 = jnp.dot(h1.astype(bf), w2, preferred_element_type=jnp.float32)
        h2 = jnp.maximum(h2 + b2_ref[...], 0.0)
        kfull = jnp.dot(h2.astype(bf), w3, preferred_element_type=jnp.float32)
        kw = (kfull[:, 0:1] + b3_ref[...]) * (1.0 / math.sqrt(_K))
        msg = kw * u_f
        agg_s = jnp.sum(msg.reshape(kc, _RC, _D), axis=0)
        agg = agg_s if agg is None else agg + agg_s
    u_blk = t[:, :_D]
    out = jnp.dot(u_blk, wl_ref[...], preferred_element_type=jnp.float32)
    out = out + bl_ref[...] + agg + u_blk
    out_ref[0] = jnp.maximum(out, 0.0)


def _mlp_call(t3, g4, wi, wgu, wgx, b1, w2, b2, w3r, b3, wl, bl):
    full = lambda s: pl.BlockSpec(s, lambda i: tuple(0 for _ in s))
    return pl.pallas_call(
        _mlp_body,
        grid=(_N // _RC,),
        in_specs=[
            pl.BlockSpec((1, _RC, _TPAD), lambda i: (0, i, 0)),
            pl.BlockSpec((1, _K, _RC, _D), lambda i: (0, 0, i, 0)),
            full((_TPAD, _H)),
            full((_D, _H)),
            full((_D, _H)),
            full((1, _H)),
            full((_H, _H)),
            full((1, _H)),
            full((_H, _D)),
            full((1, 1)),
            full((_D, _D)),
            full((1, _D)),
        ],
        out_specs=pl.BlockSpec((1, _RC, _D), lambda i: (0, i, 0)),
        out_shape=jax.ShapeDtypeStruct((1, _N, _D), jnp.float32),
    )(t3, g4, wi, wgu, wgx, b1, w2, b2, w3r, b3, wl, bl)


def kernel(u, coords, W_local, b_local, W1, b1, W2, b2, W3, b3):
    b, n, d = u.shape
    c = coords.shape[-1]
    coordst = jnp.swapaxes(coords, 1, 2)

    zpad = jnp.zeros((_TPAD - d - c, _H), jnp.float32)
    wi = jnp.concatenate([W1[2 * c:2 * c + d], W1[0:c], zpad], axis=0)
    wgu = W1[2 * c + d:]
    wgx = jnp.concatenate(
        [W1[c:2 * c], jnp.zeros((_D - c, _H), jnp.float32)], axis=0)
    w3p = jnp.concatenate([W3, jnp.zeros((_H, _D - 1), jnp.float32)], axis=1)

    outs = []
    for bi in range(b):
        idx, t3, tpack = _knn_call(
            u[bi:bi + 1], coords[bi:bi + 1], coordst[bi:bi + 1])
        idx2d = idx.reshape(n * _K // _CHUNK, _CHUNK)
        g = _gather_call(tpack.reshape(n, _D), idx2d)
        g4 = g.reshape(1, _K, n, _D)
        outs.append(_mlp_call(
            t3, g4, wi, wgu, wgx,
            b1.reshape(1, _H), W2, b2.reshape(1, _H),
            w3p, b3.reshape(1, 1),
            W_local, b_local.reshape(1, _D)))
    return jnp.concatenate(outs, axis=0)


# speedup vs baseline: 16.3914x; 1.0036x over previous
import math

import jax
import jax.numpy as jnp
from jax import lax
from jax.experimental import pallas as pl
from jax.experimental.pallas import tpu as pltpu
from jax.experimental.pallas import tpu_sc as plsc

_B, _N, _D, _C, _H, _K = 2, 2048, 128, 3, 256, 16
_TPAD = 256
_RA = 256
_RC = 256

_NC, _NS = 2, 16
_NW = _NC * _NS
_E = _B * _N * _K
_EPW = _E // _NW
_CHUNK = 128


def _knn_body(u_ref, coords_ref, coordst_ref, idx_ref, t3_ref, tp_ref):
    xi = coords_ref[0]
    xt = coordst_ref[0]

    ub = u_ref[0]
    xpad = jnp.concatenate(
        [xi, jnp.zeros((_RA, _D - _C), jnp.float32)], axis=1)
    t3_ref[0] = jnp.concatenate([ub, xpad], axis=1)
    ub16 = lax.bitcast_convert_type(ub.astype(jnp.bfloat16), jnp.uint16)
    xb16 = lax.bitcast_convert_type(xpad.astype(jnp.bfloat16), jnp.uint16)
    tp_ref[0] = lax.bitcast_convert_type(
        ub16.astype(jnp.uint32) | (xb16.astype(jnp.uint32) << 16), jnp.int32)
    d0 = xi[:, 0:1] - xt[0:1, :]
    acc = d0 * d0
    d1 = xi[:, 1:2] - xt[1:2, :]
    acc = acc + d1 * d1
    d2 = xi[:, 2:3] - xt[2:3, :]
    acc = acc + d2 * d2

    half = _N // 2
    d_a = acc[:, :half]
    d_b = acc[:, half:]
    iotah = lax.broadcasted_iota(jnp.int32, (_RA, half), 1).astype(jnp.float32)
    cond = d_b < d_a
    lo = jnp.minimum(d_a, d_b)
    hi = jnp.maximum(d_a, d_b)
    elo = jnp.where(cond, iotah + half, iotah)
    ehi = jnp.where(cond, iotah, iotah + half)
    big = jnp.float32(_N)
    inf = jnp.float32(jnp.inf)
    cols = []
    for _ in range(_K):
        m = jnp.min(lo, axis=1, keepdims=True)
        j = jnp.min(jnp.where(lo == m, elo, big), axis=1, keepdims=True)
        cols.append(j)
        pick = elo == j
        lo = jnp.where(pick, hi, lo)
        elo = jnp.where(pick, ehi, elo)
        hi = jnp.where(pick, inf, hi)
    idx = jnp.concatenate(cols, axis=1).astype(jnp.int32)
    idx_ref[0] = idx.T


def _knn_call(u, coords, coordst):
    return pl.pallas_call(
        _knn_body,
        grid=(_N // _RA,),
        in_specs=[
            pl.BlockSpec((1, _RA, _D), lambda i: (0, i, 0)),
            pl.BlockSpec((1, _RA, _C), lambda i: (0, i, 0)),
            pl.BlockSpec((1, _C, _N), lambda i: (0, 0, 0)),
        ],
        out_specs=[
            pl.BlockSpec((1, _K, _RA), lambda i: (0, 0, i)),
            pl.BlockSpec((1, _RA, _TPAD), lambda i: (0, i, 0)),
            pl.BlockSpec((1, _RA, _D), lambda i: (0, i, 0)),
        ],
        out_shape=[
            jax.ShapeDtypeStruct((1, _K, _N), jnp.int32),
            jax.ShapeDtypeStruct((1, _N, _TPAD), jnp.float32),
            jax.ShapeDtypeStruct((1, _N, _D), jnp.int32),
        ],
    )(u, coords, coordst)


def _gather_body(t_hbm, idx_hbm, out_hbm, idx_v, rv0, rv1, sem0, sem1):
    wid = lax.axis_index("s") * _NC + lax.axis_index("c")
    nchunk = idx_v.shape[0]
    row0 = wid * nchunk
    pltpu.sync_copy(idx_hbm.at[pl.ds(row0, nchunk)], idx_v)

    rv = (rv0, rv1)
    sems = (sem0, sem1)
    prev = None
    for ci in range(nchunk):
        cur = pltpu.async_copy(
            t_hbm.at[idx_v.at[ci]], rv[ci % 2], sems[ci % 2])
        if prev is not None:
            prev.wait()
            pltpu.sync_copy(
                rv[(ci - 1) % 2],
                out_hbm.at[pl.ds((row0 + ci - 1) * _CHUNK, _CHUNK)])
        prev = cur
    prev.wait()
    pltpu.sync_copy(
        rv[(nchunk - 1) % 2],
        out_hbm.at[pl.ds((row0 + nchunk - 1) * _CHUNK, _CHUNK)])


def _gather_call(t_flat, idx2d):
    mesh = plsc.VectorSubcoreMesh(
        core_axis_name="c", subcore_axis_name="s",
        num_cores=_NC, num_subcores=_NS)
    ne = idx2d.shape[0] * _CHUNK
    f = pl.kernel(
        _gather_body,
        out_type=jax.ShapeDtypeStruct((ne, _D), jnp.int32),
        mesh=mesh,
        scratch_types=[
            pltpu.VMEM((ne // _NW // _CHUNK, _CHUNK), jnp.int32),
            pltpu.VMEM((_CHUNK, _D), jnp.int32),
            pltpu.VMEM((_CHUNK, _D), jnp.int32),
            pltpu.SemaphoreType.DMA,
            pltpu.SemaphoreType.DMA,
        ],
    )
    return f(t_flat, idx2d)


def _mlp_body(t_ref, g_ref, wi_ref, wgu_ref, wgx_ref, b1_ref, w2_ref, b2_ref,
              w3_ref, b3_ref, wl_ref, bl_ref, out_ref):
    t = t_ref[0]
    bf = jnp.bfloat16
    wgu = wgu_ref[...].astype(bf)
    wgx = wgx_ref[...].astype(bf)
    w2 = w2_ref[...].astype(bf)
    w3 = w3_ref[...].astype(bf)
    a_i = jnp.dot(t.astype(bf), wi_ref[...].astype(bf),
                  preferred_element_type=jnp.float32)
    a_i = a_i + b1_ref[...]
    sub = 8
    kc = _K // sub
    rows = kc * _RC
    agg = None
    for s in range(sub):
        gs = g_ref[0, s * kc:(s + 1) * kc, :, :].reshape(rows, _D)
        u_f = lax.bitcast_convert_type(gs << 16, jnp.float32)
        x_f = lax.bitcast_convert_type(gs & jnp.int32(-65536), jnp.float32)
        a_j = (jnp.dot(u_f.astype(bf), wgu, preferred_element_type=jnp.float32)
               + jnp.dot(x_f.astype(bf), wgx,
                         preferred_element_type=jnp.float32))
        a_i_rep = jnp.broadcast_to(
            a_i[None, :, :], (kc, _RC, _H)).reshape(rows, _H)
        h1 = jnp.maximum(a_j + a_i_rep, 0.0)
        h2 = jnp.dot(h1.astype(bf), w2, preferred_element_type=jnp.float32)
        h2 = jnp.maximum(h2 + b2_ref[...], 0.0)
        kfull = jnp.dot(h2.astype(bf), w3, preferred_element_type=jnp.float32)
        kw = (kfull[:, 0:1] + b3_ref[...]) * (1.0 / math.sqrt(_K))
        msg = kw * u_f
        agg_s = jnp.sum(msg.reshape(kc, _RC, _D), axis=0)
        agg = agg_s if agg is None else agg + agg_s
    u_blk = t[:, :_D]
    out = jnp.dot(u_blk, wl_ref[...], preferred_element_type=jnp.float32)
    out = out + bl_ref[...] + agg + u_blk
    out_ref[0] = jnp.maximum(out, 0.0)


def _mlp_call(t3, g4, wi, wgu, wgx, b1, w2, b2, w3r, b3, wl, bl):
    full = lambda s: pl.BlockSpec(s, lambda i: tuple(0 for _ in s))
    return pl.pallas_call(
        _mlp_body,
        grid=(_N // _RC,),
        in_specs=[
            pl.BlockSpec((1, _RC, _TPAD), lambda i: (0, i, 0)),
            pl.BlockSpec((1, _K, _RC, _D), lambda i: (0, 0, i, 0)),
            full((_TPAD, _H)),
            full((_D, _H)),
            full((_D, _H)),
            full((1, _H)),
            full((_H, _H)),
            full((1, _H)),
            full((_H, _D)),
            full((1, 1)),
            full((_D, _D)),
            full((1, _D)),
        ],
        out_specs=pl.BlockSpec((1, _RC, _D), lambda i: (0, i, 0)),
        out_shape=jax.ShapeDtypeStruct((1, _N, _D), jnp.float32),
    )(t3, g4, wi, wgu, wgx, b1, w2, b2, w3r, b3, wl, bl)


def kernel(u, coords, W_local, b_local, W1, b1, W2, b2, W3, b3):
    b, n, d = u.shape
    c = coords.shape[-1]
    coordst = jnp.swapaxes(coords, 1, 2)

    zpad = jnp.zeros((_TPAD - d - c, _H), jnp.float32)
    wi = jnp.concatenate([W1[2 * c:2 * c + d], W1[0:c], zpad], axis=0)
    wgu = W1[2 * c + d:]
    wgx = jnp.concatenate(
        [W1[c:2 * c], jnp.zeros((_D - c, _H), jnp.float32)], axis=0)
    w3p = jnp.concatenate([W3, jnp.zeros((_H, _D - 1), jnp.float32)], axis=1)

    outs = []
    for bi in range(b):
        idx, t3, tpack = _knn_call(
            u[bi:bi + 1], coords[bi:bi + 1], coordst[bi:bi + 1])
        idx2d = idx.reshape(n * _K // _CHUNK, _CHUNK)
        g = _gather_call(tpack.reshape(n, _D), idx2d)
        g4 = g.reshape(1, _K, n, _D)
        outs.append(_mlp_call(
            t3, g4, wi, wgu, wgx,
            b1.reshape(1, _H), W2, b2.reshape(1, _H),
            w3p, b3.reshape(1, 1),
            W_local, b_local.reshape(1, _D)))
    return jnp.concatenate(outs, axis=0)
